# Initial kernel scaffold; baseline (speedup 1.0000x reference)
import jax
import jax.numpy as jnp
from jax.experimental import pallas as pl


def kernel(positions):
    raise NotImplementedError("write your pallas kernel here")


# speedup vs baseline: 2.5153x; 2.5153x over previous
import jax
import jax.numpy as jnp
from jax.experimental import pallas as pl
from jax.experimental.pallas import tpu as pltpu

N_ATOMS = 4096
CUTOFF = 10.0
MAX_NEIGHBORS = 50

_BR = 128
_L = 128


def _lex_lt(av, ai, bv, bi):
    return (av < bv) | ((av == bv) & (ai < bi))


def _stage(v, i, lane, d, desc):
    upper = (lane & d) != 0
    pv = jnp.where(upper, pltpu.roll(v, d, 1), pltpu.roll(v, _L - d, 1))
    pi = jnp.where(upper, pltpu.roll(i, d, 1), pltpu.roll(i, _L - d, 1))
    lt = _lex_lt(pv, pi, v, i)
    take = lt ^ upper ^ desc
    v = jnp.where(take, pv, v)
    i = jnp.where(take, pi, i)
    return v, i


def _topk_kernel(pos_ref, post_ref, vals_ref, idxs_ref, accv_ref, acci_ref):
    r = pl.program_id(0)
    c = pl.program_id(1)
    nc = pl.num_programs(1)

    pos_blk = pos_ref[...]
    xj = post_ref[0:1, :]
    yj = post_ref[1:2, :]
    zj = post_ref[2:3, :]

    sq_i = jnp.sum(pos_blk * pos_blk, axis=1, keepdims=True)
    sq_j = xj * xj + yj * yj + zj * zj
    dot = jnp.dot(pos_blk, post_ref[:3, :],
                  preferred_element_type=jnp.float32)
    dist2 = (sq_i + sq_j) - 2.0 * dot
    dist2 = jnp.maximum(dist2, 0.0)

    lane = jax.lax.broadcasted_iota(jnp.int32, (_BR, _L), 1)
    col = lane + c * _L
    row = jax.lax.broadcasted_iota(jnp.int32, (_BR, _L), 0) + r * _BR
    dist2 = dist2 + jnp.where(col == row, 1e12, 0.0)

    dist = jnp.sqrt(dist2)
    v = jnp.where(dist <= CUTOFF, dist, dist + 1e6)
    i = col

    k = 2
    while k <= _L:
        blockdesc = (lane & k) == 0 if k < _L else True
        d = k // 2
        while d >= 1:
            v, i = _stage(v, i, lane, d, blockdesc)
            d //= 2
        k *= 2

    @pl.when(c == 0)
    def _init():
        accv_ref[...] = jnp.full((_BR, _L), 3.0e38, dtype=jnp.float32)
        acci_ref[...] = jnp.zeros((_BR, _L), dtype=jnp.int32)

    av, ai = accv_ref[...], acci_ref[...]
    lt = _lex_lt(av, ai, v, i)
    v = jnp.where(lt, av, v)
    i = jnp.where(lt, ai, i)
    d = _L // 2
    while d >= 1:
        v, i = _stage(v, i, lane, d, False)
        d //= 2
    accv_ref[...] = v
    acci_ref[...] = i

    @pl.when(c == nc - 1)
    def _emit():
        vals_ref[...] = v
        idxs_ref[...] = i


@jax.jit
def kernel(positions):
    n = positions.shape[0]
    pos_t = jnp.zeros((8, n), dtype=positions.dtype).at[:3, :].set(positions.T)

    vals, idxs = pl.pallas_call(
        _topk_kernel,
        grid=(n // _BR, n // _L),
        in_specs=[
            pl.BlockSpec((_BR, 3), lambda r, c: (r, 0)),
            pl.BlockSpec((8, _L), lambda r, c: (0, c)),
        ],
        out_specs=[
            pl.BlockSpec((_BR, _L), lambda r, c: (r, 0)),
            pl.BlockSpec((_BR, _L), lambda r, c: (r, 0)),
        ],
        out_shape=[
            jax.ShapeDtypeStruct((n, _L), positions.dtype),
            jax.ShapeDtypeStruct((n, _L), jnp.int32),
        ],
        scratch_shapes=[
            pltpu.VMEM((_BR, _L), jnp.float32),
            pltpu.VMEM((_BR, _L), jnp.int32),
        ],
    )(positions, pos_t)

    nbr_dist = vals[:, :MAX_NEIGHBORS]
    nbr_idx = idxs[:, :MAX_NEIGHBORS]

    edge_distance = jnp.where(nbr_dist <= CUTOFF, nbr_dist, 0.0).reshape(-1)
    c_index = jnp.repeat(jnp.arange(n, dtype=jnp.int32), MAX_NEIGHBORS)
    n_index = nbr_idx.reshape(-1)
    edge_index = jnp.stack([n_index, c_index], axis=0)
    offsets = jnp.zeros((n * MAX_NEIGHBORS, 3), dtype=positions.dtype)
    return edge_index, edge_distance, offsets


# speedup vs baseline: 3.2268x; 1.2828x over previous
import jax
import jax.numpy as jnp
from jax.experimental import pallas as pl
from jax.experimental.pallas import tpu as pltpu

N_ATOMS = 4096
CUTOFF = 10.0
MAX_NEIGHBORS = 50

_BR = 128
_L = 128
_CPS = 4


def _lex_lt(av, ai, bv, bi):
    return (av < bv) | ((av == bv) & (ai < bi))


def _stage(v, i, lane, d, desc):
    upper = (lane & d) != 0
    pv = jnp.where(upper, pltpu.roll(v, d, 1), pltpu.roll(v, _L - d, 1))
    pi = jnp.where(upper, pltpu.roll(i, d, 1), pltpu.roll(i, _L - d, 1))
    lt = _lex_lt(pv, pi, v, i)
    take = lt ^ upper ^ desc
    v = jnp.where(take, pv, v)
    i = jnp.where(take, pi, i)
    return v, i


def _sort_lanes(v, i, lane, descending):
    k = 2
    while k <= _L:
        if k < _L:
            blockdesc = ((lane & k) == 0) if descending else ((lane & k) != 0)
        else:
            blockdesc = descending
        d = k // 2
        while d >= 1:
            v, i = _stage(v, i, lane, d, blockdesc)
            d //= 2
        k *= 2
    return v, i


def _fold(av, ai, bv, bi):
    lt = _lex_lt(av, ai, bv, bi)
    return jnp.where(lt, av, bv), jnp.where(lt, ai, bi)


def _clean(v, i, lane, descending):
    d = _L // 2
    while d >= 1:
        v, i = _stage(v, i, lane, d, descending)
        d //= 2
    return v, i


def _topk_kernel(pos_ref, post_ref, vals_ref, idxs_ref, accv_ref, acci_ref):
    r = pl.program_id(0)
    c = pl.program_id(1)
    nc = pl.num_programs(1)
    w = _CPS * _L

    pos_blk = pos_ref[...]
    xj = post_ref[0:1, :]
    yj = post_ref[1:2, :]
    zj = post_ref[2:3, :]

    sq_i = jnp.sum(pos_blk * pos_blk, axis=1, keepdims=True)
    sq_j = xj * xj + yj * yj + zj * zj
    dot = jnp.dot(pos_blk, post_ref[:3, :],
                  preferred_element_type=jnp.float32)
    dist2 = (sq_i + sq_j) - 2.0 * dot
    dist2 = jnp.maximum(dist2, 0.0)

    colw = jax.lax.broadcasted_iota(jnp.int32, (_BR, w), 1) + c * w
    roww = jax.lax.broadcasted_iota(jnp.int32, (_BR, w), 0) + r * _BR
    dist2 = dist2 + jnp.where(colw == roww, 1e12, 0.0)

    dist = jnp.sqrt(dist2)
    keyed = jnp.where(dist <= CUTOFF, dist, dist + 1e6)

    lane = jax.lax.broadcasted_iota(jnp.int32, (_BR, _L), 1)

    vs = [keyed[:, j * _L:(j + 1) * _L] for j in range(_CPS)]
    js = [colw[:, j * _L:(j + 1) * _L] for j in range(_CPS)]
    descs = [bool(j % 2) for j in range(_CPS)]
    k = 2
    while k <= _L:
        d = k // 2
        while d >= 1:
            for j in range(_CPS):
                if k < _L:
                    bd = ((lane & k) == 0) if descs[j] else ((lane & k) != 0)
                else:
                    bd = descs[j]
                vs[j], js[j] = _stage(vs[j], js[j], lane, d, bd)
            d //= 2
        k *= 2

    v01, i01 = _fold(vs[0], js[0], vs[1], js[1])
    v23, i23 = _fold(vs[2], js[2], vs[3], js[3])
    d = _L // 2
    while d >= 1:
        v01, i01 = _stage(v01, i01, lane, d, False)
        v23, i23 = _stage(v23, i23, lane, d, True)
        d //= 2
    v, i = _fold(v01, i01, v23, i23)
    v, i = _clean(v, i, lane, descending=True)

    @pl.when(c == 0)
    def _init():
        accv_ref[...] = jnp.full((_BR, _L), 3.0e38, dtype=jnp.float32)
        acci_ref[...] = jnp.zeros((_BR, _L), dtype=jnp.int32)

    av, ai = accv_ref[...], acci_ref[...]
    v, i = _fold(av, ai, v, i)
    v, i = _clean(v, i, lane, descending=False)
    accv_ref[...] = v
    acci_ref[...] = i

    @pl.when(c == nc - 1)
    def _emit():
        vals_ref[...] = v
        idxs_ref[...] = i


@jax.jit
def kernel(positions):
    n = positions.shape[0]
    pos_t = jnp.zeros((8, n), dtype=positions.dtype).at[:3, :].set(positions.T)

    vals, idxs = pl.pallas_call(
        _topk_kernel,
        grid=(n // _BR, n // (_CPS * _L)),
        in_specs=[
            pl.BlockSpec((_BR, 3), lambda r, c: (r, 0)),
            pl.BlockSpec((8, _CPS * _L), lambda r, c: (0, c)),
        ],
        out_specs=[
            pl.BlockSpec((_BR, _L), lambda r, c: (r, 0)),
            pl.BlockSpec((_BR, _L), lambda r, c: (r, 0)),
        ],
        out_shape=[
            jax.ShapeDtypeStruct((n, _L), positions.dtype),
            jax.ShapeDtypeStruct((n, _L), jnp.int32),
        ],
        scratch_shapes=[
            pltpu.VMEM((_BR, _L), jnp.float32),
            pltpu.VMEM((_BR, _L), jnp.int32),
        ],
    )(positions, pos_t)

    nbr_dist = vals[:, :MAX_NEIGHBORS]
    nbr_idx = idxs[:, :MAX_NEIGHBORS]

    edge_distance = jnp.where(nbr_dist <= CUTOFF, nbr_dist, 0.0).reshape(-1)
    c_index = jnp.repeat(jnp.arange(n, dtype=jnp.int32), MAX_NEIGHBORS)
    n_index = nbr_idx.reshape(-1)
    edge_index = jnp.stack([n_index, c_index], axis=0)
    offsets = jnp.zeros((n * MAX_NEIGHBORS, 3), dtype=positions.dtype)
    return edge_index, edge_distance, offsets


# speedup vs baseline: 3.4431x; 1.0670x over previous
import jax
import jax.numpy as jnp
from jax.experimental import pallas as pl
from jax.experimental.pallas import tpu as pltpu

N_ATOMS = 4096
CUTOFF = 10.0
MAX_NEIGHBORS = 50

_BR = 128
_L = 128
_CPS = 4


def _lex_lt(av, ai, bv, bi):
    return (av < bv) | ((av == bv) & (ai < bi))


def _stage(v, i, lane, d, desc):
    upper = (lane & d) != 0
    pv = jnp.where(upper, pltpu.roll(v, d, 1), pltpu.roll(v, _L - d, 1))
    pi = jnp.where(upper, pltpu.roll(i, d, 1), pltpu.roll(i, _L - d, 1))
    lt = _lex_lt(pv, pi, v, i)
    take = lt ^ upper ^ desc
    v = jnp.where(take, pv, v)
    i = jnp.where(take, pi, i)
    return v, i


def _sort_lanes(v, i, lane, descending):
    k = 2
    while k <= _L:
        if k < _L:
            blockdesc = ((lane & k) == 0) if descending else ((lane & k) != 0)
        else:
            blockdesc = descending
        d = k // 2
        while d >= 1:
            v, i = _stage(v, i, lane, d, blockdesc)
            d //= 2
        k *= 2
    return v, i


def _fold(av, ai, bv, bi):
    lt = _lex_lt(av, ai, bv, bi)
    return jnp.where(lt, av, bv), jnp.where(lt, ai, bi)


def _clean(v, i, lane, descending):
    d = _L // 2
    while d >= 1:
        v, i = _stage(v, i, lane, d, descending)
        d //= 2
    return v, i


def _topk_kernel(pos_ref, post_ref, vals_ref, idxs_ref, accv_ref, acci_ref):
    r = pl.program_id(0)
    c = pl.program_id(1)
    nc = pl.num_programs(1)
    w = _CPS * _L

    pos_blk = pos_ref[...]
    xj = post_ref[0:1, :]
    yj = post_ref[1:2, :]
    zj = post_ref[2:3, :]

    sq_i = jnp.sum(pos_blk * pos_blk, axis=1, keepdims=True)
    sq_j = xj * xj + yj * yj + zj * zj
    dot = jnp.dot(pos_blk, post_ref[:3, :],
                  preferred_element_type=jnp.float32)
    dist2 = (sq_i + sq_j) - 2.0 * dot
    dist2 = jnp.maximum(dist2, 0.0)

    colw = jax.lax.broadcasted_iota(jnp.int32, (_BR, w), 1) + c * w
    roww = jax.lax.broadcasted_iota(jnp.int32, (_BR, w), 0) + r * _BR
    dist2 = dist2 + jnp.where(colw == roww, 1e12, 0.0)

    dist = jnp.sqrt(dist2)
    keyed = jnp.where(dist <= CUTOFF, dist, dist + 1e6)

    lane = jax.lax.broadcasted_iota(jnp.int32, (_BR, _L), 1)

    vs = [keyed[:, j * _L:(j + 1) * _L] for j in range(_CPS)]
    js = [colw[:, j * _L:(j + 1) * _L] for j in range(_CPS)]
    descs = [bool(j % 2) for j in range(_CPS)]
    k = 2
    while k <= _L:
        d = k // 2
        while d >= 1:
            for j in range(_CPS):
                if k < _L:
                    bd = ((lane & k) == 0) if descs[j] else ((lane & k) != 0)
                else:
                    bd = descs[j]
                vs[j], js[j] = _stage(vs[j], js[j], lane, d, bd)
            d //= 2
        k *= 2

    v01, i01 = _fold(vs[0], js[0], vs[1], js[1])
    v23, i23 = _fold(vs[2], js[2], vs[3], js[3])
    d = _L // 2
    while d >= 1:
        v01, i01 = _stage(v01, i01, lane, d, True)
        v23, i23 = _stage(v23, i23, lane, d, False)
        d //= 2

    @pl.when(c == 0)
    def _init():
        accv_ref[...] = jnp.full((2 * _BR, _L), 3.0e38, dtype=jnp.float32)
        acci_ref[...] = jnp.zeros((2 * _BR, _L), dtype=jnp.int32)

    av, ai = accv_ref[:_BR, :], acci_ref[:_BR, :]
    bv, bi = accv_ref[_BR:, :], acci_ref[_BR:, :]
    av, ai = _fold(av, ai, v01, i01)
    bv, bi = _fold(bv, bi, v23, i23)
    d = _L // 2
    while d >= 1:
        av, ai = _stage(av, ai, lane, d, False)
        bv, bi = _stage(bv, bi, lane, d, True)
        d //= 2
    accv_ref[:_BR, :], acci_ref[:_BR, :] = av, ai
    accv_ref[_BR:, :], acci_ref[_BR:, :] = bv, bi

    @pl.when(c == nc - 1)
    def _emit():
        fv, fi = _fold(av, ai, bv, bi)
        d = _L // 2
        while d >= 1:
            fv, fi = _stage(fv, fi, lane, d, False)
            d //= 2
        vals_ref[...] = fv
        idxs_ref[...] = fi


@jax.jit
def kernel(positions):
    n = positions.shape[0]
    pos_t = jnp.zeros((8, n), dtype=positions.dtype).at[:3, :].set(positions.T)

    vals, idxs = pl.pallas_call(
        _topk_kernel,
        grid=(n // _BR, n // (_CPS * _L)),
        in_specs=[
            pl.BlockSpec((_BR, 3), lambda r, c: (r, 0)),
            pl.BlockSpec((8, _CPS * _L), lambda r, c: (0, c)),
        ],
        out_specs=[
            pl.BlockSpec((_BR, _L), lambda r, c: (r, 0)),
            pl.BlockSpec((_BR, _L), lambda r, c: (r, 0)),
        ],
        out_shape=[
            jax.ShapeDtypeStruct((n, _L), positions.dtype),
            jax.ShapeDtypeStruct((n, _L), jnp.int32),
        ],
        scratch_shapes=[
            pltpu.VMEM((2 * _BR, _L), jnp.float32),
            pltpu.VMEM((2 * _BR, _L), jnp.int32),
        ],
    )(positions, pos_t)

    nbr_dist = vals[:, :MAX_NEIGHBORS]
    nbr_idx = idxs[:, :MAX_NEIGHBORS]

    edge_distance = jnp.where(nbr_dist <= CUTOFF, nbr_dist, 0.0).reshape(-1)
    c_index = jnp.repeat(jnp.arange(n, dtype=jnp.int32), MAX_NEIGHBORS)
    n_index = nbr_idx.reshape(-1)
    edge_index = jnp.stack([n_index, c_index], axis=0)
    offsets = jnp.zeros((n * MAX_NEIGHBORS, 3), dtype=positions.dtype)
    return edge_index, edge_distance, offsets


# speedup vs baseline: 3.4893x; 1.0134x over previous
import jax
import jax.numpy as jnp
from jax.experimental import pallas as pl
from jax.experimental.pallas import tpu as pltpu

N_ATOMS = 4096
CUTOFF = 10.0
MAX_NEIGHBORS = 50

_BR = 128
_L = 128
_CPS = 8


def _lex_lt(av, ai, bv, bi):
    return (av < bv) | ((av == bv) & (ai < bi))


def _stage(v, i, lane, d, desc):
    upper = (lane & d) != 0
    pv = jnp.where(upper, pltpu.roll(v, d, 1), pltpu.roll(v, _L - d, 1))
    pi = jnp.where(upper, pltpu.roll(i, d, 1), pltpu.roll(i, _L - d, 1))
    lt = _lex_lt(pv, pi, v, i)
    take = lt ^ upper ^ desc
    v = jnp.where(take, pv, v)
    i = jnp.where(take, pi, i)
    return v, i


def _sort_lanes(v, i, lane, descending):
    k = 2
    while k <= _L:
        if k < _L:
            blockdesc = ((lane & k) == 0) if descending else ((lane & k) != 0)
        else:
            blockdesc = descending
        d = k // 2
        while d >= 1:
            v, i = _stage(v, i, lane, d, blockdesc)
            d //= 2
        k *= 2
    return v, i


def _fold(av, ai, bv, bi):
    lt = _lex_lt(av, ai, bv, bi)
    return jnp.where(lt, av, bv), jnp.where(lt, ai, bi)


def _clean(v, i, lane, descending):
    d = _L // 2
    while d >= 1:
        v, i = _stage(v, i, lane, d, descending)
        d //= 2
    return v, i


def _topk_kernel(pos_ref, post_ref, vals_ref, idxs_ref, accv_ref, acci_ref):
    r = pl.program_id(0)
    c = pl.program_id(1)
    nc = pl.num_programs(1)
    w = _CPS * _L

    pos_blk = pos_ref[...]
    xj = post_ref[0:1, :]
    yj = post_ref[1:2, :]
    zj = post_ref[2:3, :]

    sq_i = jnp.sum(pos_blk * pos_blk, axis=1, keepdims=True)
    sq_j = xj * xj + yj * yj + zj * zj
    dot = jnp.dot(pos_blk, post_ref[:3, :],
                  preferred_element_type=jnp.float32)
    dist2 = (sq_i + sq_j) - 2.0 * dot
    dist2 = jnp.maximum(dist2, 0.0)

    colw = jax.lax.broadcasted_iota(jnp.int32, (_BR, w), 1) + c * w
    roww = jax.lax.broadcasted_iota(jnp.int32, (_BR, w), 0) + r * _BR
    dist2 = dist2 + jnp.where(colw == roww, 1e12, 0.0)

    dist = jnp.sqrt(dist2)
    keyed = jnp.where(dist <= CUTOFF, dist, dist + 1e6)

    lane = jax.lax.broadcasted_iota(jnp.int32, (_BR, _L), 1)

    vs = [keyed[:, j * _L:(j + 1) * _L] for j in range(_CPS)]
    js = [colw[:, j * _L:(j + 1) * _L] for j in range(_CPS)]
    descs = [bool(j % 2) for j in range(_CPS)]
    k = 2
    while k <= _L:
        d = k // 2
        while d >= 1:
            for j in range(_CPS):
                if k < _L:
                    bd = ((lane & k) == 0) if descs[j] else ((lane & k) != 0)
                else:
                    bd = descs[j]
                vs[j], js[j] = _stage(vs[j], js[j], lane, d, bd)
            d //= 2
        k *= 2

    while len(vs) > 2:
        nv, ni, ndesc = [], [], []
        for t in range(len(vs) // 2):
            fv, fi = _fold(vs[2 * t], js[2 * t], vs[2 * t + 1], js[2 * t + 1])
            nv.append(fv)
            ni.append(fi)
            ndesc.append(t % 2 == 0)
        d = _L // 2
        while d >= 1:
            for t in range(len(nv)):
                nv[t], ni[t] = _stage(nv[t], ni[t], lane, d, ndesc[t])
            d //= 2
        vs, js = nv, ni
    v01, i01 = vs[0], js[0]
    v23, i23 = vs[1], js[1]

    @pl.when(c == 0)
    def _init():
        accv_ref[...] = jnp.full((2 * _BR, _L), 3.0e38, dtype=jnp.float32)
        acci_ref[...] = jnp.zeros((2 * _BR, _L), dtype=jnp.int32)

    av, ai = accv_ref[:_BR, :], acci_ref[:_BR, :]
    bv, bi = accv_ref[_BR:, :], acci_ref[_BR:, :]
    av, ai = _fold(av, ai, v01, i01)
    bv, bi = _fold(bv, bi, v23, i23)
    d = _L // 2
    while d >= 1:
        av, ai = _stage(av, ai, lane, d, False)
        bv, bi = _stage(bv, bi, lane, d, True)
        d //= 2
    accv_ref[:_BR, :], acci_ref[:_BR, :] = av, ai
    accv_ref[_BR:, :], acci_ref[_BR:, :] = bv, bi

    @pl.when(c == nc - 1)
    def _emit():
        fv, fi = _fold(av, ai, bv, bi)
        d = _L // 2
        while d >= 1:
            fv, fi = _stage(fv, fi, lane, d, False)
            d //= 2
        vals_ref[...] = fv
        idxs_ref[...] = fi


@jax.jit
def kernel(positions):
    n = positions.shape[0]
    pos_t = jnp.zeros((8, n), dtype=positions.dtype).at[:3, :].set(positions.T)

    vals, idxs = pl.pallas_call(
        _topk_kernel,
        grid=(n // _BR, n // (_CPS * _L)),
        in_specs=[
            pl.BlockSpec((_BR, 3), lambda r, c: (r, 0)),
            pl.BlockSpec((8, _CPS * _L), lambda r, c: (0, c)),
        ],
        out_specs=[
            pl.BlockSpec((_BR, _L), lambda r, c: (r, 0)),
            pl.BlockSpec((_BR, _L), lambda r, c: (r, 0)),
        ],
        out_shape=[
            jax.ShapeDtypeStruct((n, _L), positions.dtype),
            jax.ShapeDtypeStruct((n, _L), jnp.int32),
        ],
        scratch_shapes=[
            pltpu.VMEM((2 * _BR, _L), jnp.float32),
            pltpu.VMEM((2 * _BR, _L), jnp.int32),
        ],
    )(positions, pos_t)

    nbr_dist = vals[:, :MAX_NEIGHBORS]
    nbr_idx = idxs[:, :MAX_NEIGHBORS]

    edge_distance = jnp.where(nbr_dist <= CUTOFF, nbr_dist, 0.0).reshape(-1)
    c_index = jnp.repeat(jnp.arange(n, dtype=jnp.int32), MAX_NEIGHBORS)
    n_index = nbr_idx.reshape(-1)
    edge_index = jnp.stack([n_index, c_index], axis=0)
    offsets = jnp.zeros((n * MAX_NEIGHBORS, 3), dtype=positions.dtype)
    return edge_index, edge_distance, offsets
